# baseline (speedup 1.0000x reference)
import jax
import jax.numpy as jnp
from jax import lax
from jax.experimental import pallas as pl
from jax.experimental.pallas import tpu as pltpu
from jax.experimental.pallas import tpu_sc as plsc

B = 16384
D = 64
L = 16
NC = 2
NS = 16
NW = NC * NS
BPW = B // NW
PASS = 128
NPASS = BPW // PASS
NGP = PASS // L


def _mf_body(uid_hbm, mid_hbm, uemb_hbm, memb_hbm, ubias_hbm, mbias_hbm,
             out_hbm,
             uid_v, mid_v, urows, mrows, ub2, mb2, out_v, sem, bsem):
    wid = lax.axis_index("s") * NC + lax.axis_index("c")
    base = wid * BPW

    pltpu.sync_copy(uid_hbm.at[pl.ds(base, BPW)], uid_v)
    pltpu.sync_copy(mid_hbm.at[pl.ds(base, BPW)], mid_v)

    lanes = lax.iota(jnp.int32, L)
    zeros16 = jnp.zeros((L,), jnp.int32)

    for p in range(NPASS):
        pbase = p * PASS

        def fire_body(g, carry):
            row0 = pbase + g * L
            uidv = uid_v[pl.ds(row0, L)]
            midv = mid_v[pl.ds(row0, L)]
            for j in range(L):
                uid_s = jnp.sum(jnp.where(lanes == j, uidv, 0))
                mid_s = jnp.sum(jnp.where(lanes == j, midv, 0))
                rr = g * L + j
                pltpu.async_copy(uemb_hbm.at[pl.ds(uid_s, 1)],
                                 urows.at[pl.ds(rr, 1)], sem)
                pltpu.async_copy(memb_hbm.at[pl.ds(mid_s, 1)],
                                 mrows.at[pl.ds(rr, 1)], sem)
                pltpu.async_copy(ubias_hbm.at[pl.ds(uid_s, 1)],
                                 ub2.at[pl.ds(rr, 1)], bsem)
                pltpu.async_copy(mbias_hbm.at[pl.ds(mid_s, 1)],
                                 mb2.at[pl.ds(rr, 1)], bsem)
            return carry

        lax.fori_loop(0, NGP, fire_body, 0)

        pltpu.make_async_copy(uemb_hbm.at[pl.ds(0, PASS)], urows, sem).wait()
        pltpu.make_async_copy(memb_hbm.at[pl.ds(0, PASS)], mrows, sem).wait()
        pltpu.make_async_copy(ubias_hbm.at[pl.ds(0, PASS)], ub2, bsem).wait()
        pltpu.make_async_copy(mbias_hbm.at[pl.ds(0, PASS)], mb2, bsem).wait()

        def dot_body(g, carry):
            acc = jnp.zeros((L,), jnp.float32)
            for j in range(L):
                r = g * L + j
                dotv = jnp.zeros((L,), jnp.float32)
                for q in range(D // L):
                    dsq = pl.ds(q * L, L)
                    dotv = dotv + urows[r, dsq] * mrows[r, dsq]
                acc = jnp.where(lanes == j, jnp.sum(dotv), acc)
            rows16 = g * L + lanes
            ubv = plsc.load_gather(ub2, [rows16, zeros16])
            mbv = plsc.load_gather(mb2, [rows16, zeros16])
            out_v[pl.ds(pbase + g * L, L)] = acc + ubv + mbv
            return carry

        lax.fori_loop(0, NGP, dot_body, 0)

    pltpu.sync_copy(out_v, out_hbm.at[pl.ds(base, BPW)])


@jax.jit
def _mf_call(user_ids, movie_ids, user_emb, movie_emb, user_bias, movie_bias):
    mesh = plsc.VectorSubcoreMesh(core_axis_name="c", subcore_axis_name="s")
    run = pl.kernel(
        _mf_body,
        mesh=mesh,
        compiler_params=pltpu.CompilerParams(
            needs_layout_passes=False,
            use_tc_tiling_on_sc=True,
        ),
        out_type=jax.ShapeDtypeStruct((B,), jnp.float32),
        scratch_types=[
            pltpu.VMEM((BPW,), jnp.int32),
            pltpu.VMEM((BPW,), jnp.int32),
            pltpu.VMEM((PASS, D), jnp.float32),
            pltpu.VMEM((PASS, D), jnp.float32),
            pltpu.VMEM((PASS, 1), jnp.float32),
            pltpu.VMEM((PASS, 1), jnp.float32),
            pltpu.VMEM((BPW,), jnp.float32),
            pltpu.SemaphoreType.DMA,
            pltpu.SemaphoreType.DMA,
        ],
    )
    return run(user_ids, movie_ids, user_emb, movie_emb, user_bias, movie_bias)


def kernel(user_ids, movie_ids, user_emb, movie_emb, user_bias, movie_bias):
    return _mf_call(
        user_ids.astype(jnp.int32),
        movie_ids.astype(jnp.int32),
        user_emb,
        movie_emb,
        user_bias,
        movie_bias,
    )
